# baseline (speedup 1.0000x reference)
import functools

import jax
import jax.numpy as jnp
from jax import lax
from jax.experimental import pallas as pl
from jax.experimental.pallas import tpu as pltpu
from jax.experimental.pallas import tpu_sc as plsc

L = 16
NC = 2
NS = 16
NW = NC * NS
G = 128
NB = 6


@functools.partial(jax.jit, static_argnums=(3, 4))
def _lookup(idx, first_table, second_table, n_per_w, slice_rows):
    dim = first_table.shape[1]
    n_total = idx.shape[0]
    ns = 50
    nb = n_total // ns
    buf_n = n_per_w + G

    mesh = plsc.VectorSubcoreMesh(core_axis_name="c", subcore_axis_name="s")

    @functools.partial(
        pl.kernel,
        out_type=jax.ShapeDtypeStruct((n_total, dim), jnp.float32),
        mesh=mesh,
        compiler_params=pltpu.CompilerParams(
            use_tc_tiling_on_sc=False, needs_layout_passes=False),
        scratch_types=dict(
            idx_v=pltpu.VMEM((n_per_w,), jnp.int32),
            idx1_v=pltpu.VMEM((buf_n,), jnp.int32),
            pos1_v=pltpu.VMEM((buf_n,), jnp.int32),
            idx2_v=pltpu.VMEM((buf_n,), jnp.int32),
            pos2_v=pltpu.VMEM((buf_n,), jnp.int32),
            rows_v=pltpu.VMEM((NB, G, dim), jnp.float32),
            sem_g=pltpu.SemaphoreType.DMA((NB,)),
            sem_s=pltpu.SemaphoreType.DMA((NB,)),
        ),
    )
    def k(idx_hbm, ft_hbm, st_hbm, out_hbm, *, idx_v, idx1_v, pos1_v,
          idx2_v, pos2_v, rows_v, sem_g, sem_s):
        wid = lax.axis_index("s") * NC + lax.axis_index("c")
        base = wid * n_per_w
        pltpu.sync_copy(idx_hbm.at[pl.ds(base, n_per_w)], idx_v)
        lanes = lax.iota(jnp.int32, L)

        trash = jnp.full((L,), buf_n - 1, jnp.int32)

        def half(i, o1, k):
            v = idx_v[pl.ds(i * 2 * L + k * L, L)]
            m1 = v < slice_rows
            c1 = plsc.cumsum(m1.astype(jnp.int32))
            f = base + i * 2 * L + k * L + lanes
            fb = f // ns
            p = (f - fb * ns) * nb + fb
            off1 = jnp.where(m1, o1 + c1 - 1, trash)
            off2 = jnp.where(m1, trash, (i * 2 * L + k * L - o1)
                             + (lanes - c1))
            plsc.store_scatter(idx1_v, [off1], v)
            plsc.store_scatter(pos1_v, [off1], p)
            plsc.store_scatter(idx2_v, [off2], v - slice_rows)
            plsc.store_scatter(pos2_v, [off2], p)
            return o1 + c1[L - 1]

        def step(i, o1):
            o1 = half(i, o1, 0)
            return half(i, o1, 1)

        n1 = lax.fori_loop(0, n_per_w // (2 * L), step, jnp.int32(0))
        n2 = n_per_w - n1

        def pad_dups(n, i_ref, p_ref):
            @pl.when(n > 0)
            def _():
                last = jnp.full((L,), n - 1, jnp.int32)
                li = plsc.load_gather(i_ref, [last])
                lp = plsc.load_gather(p_ref, [last])
                for t in range(G // L):
                    i_ref[pl.ds(n + t * L, L)] = li
                    p_ref[pl.ds(n + t * L, L)] = lp

        pad_dups(n1, idx1_v, pos1_v)
        pad_dups(n2, idx2_v, pos2_v)

        c1n = (n1 + G - 1) // G
        ct = c1n + (n2 + G - 1) // G

        def issue_gather(t, b):
            @pl.when(t < c1n)
            def _():
                pltpu.async_copy(
                    ft_hbm.at[idx1_v.at[pl.ds(t * G, G)]],
                    rows_v.at[b], sem_g.at[b])

            @pl.when(jnp.logical_and(t >= c1n, t < ct))
            def _():
                pltpu.async_copy(
                    st_hbm.at[idx2_v.at[pl.ds((t - c1n) * G, G)]],
                    rows_v.at[b], sem_g.at[b])

        def issue_scatter(t, b):
            @pl.when(t < c1n)
            def _():
                pltpu.async_copy(
                    rows_v.at[b], out_hbm.at[pos1_v.at[pl.ds(t * G, G)]],
                    sem_s.at[b])

            @pl.when(jnp.logical_and(t >= c1n, t < ct))
            def _():
                pltpu.async_copy(
                    rows_v.at[b],
                    out_hbm.at[pos2_v.at[pl.ds((t - c1n) * G, G)]],
                    sem_s.at[b])

        def drain(sem_b):
            pltpu.make_async_copy(
                ft_hbm.at[pl.ds(0, G)], rows_v.at[0], sem_b).wait()

        for b in range(NB):
            issue_gather(jnp.int32(b), b)

        def round_body(r, _):
            t0 = r * NB
            for b in range(NB):
                t = t0 + b

                @pl.when(t < ct)
                def _():
                    drain(sem_g.at[b])
                issue_scatter(t, b)
            for b in range(NB):
                tn = t0 + b + NB

                @pl.when(tn < ct)
                def _():
                    drain(sem_s.at[b])
                issue_gather(tn, b)
            return 0

        rounds = (ct + NB - 1) // NB
        lax.fori_loop(0, rounds, round_body, 0)

        for b in range(NB):
            @pl.when(b < ct)
            def _():
                drain(sem_s.at[b])

    return k(idx, first_table, second_table)


def kernel(batch, first_table, second_table):
    idx = batch.reshape(-1).astype(jnp.int32)
    n_total = idx.shape[0]
    assert n_total % (NW * L) == 0
    out = _lookup(idx, first_table, second_table, n_total // NW,
                  int(first_table.shape[0]))
    ns = batch.shape[1]
    x = out.reshape(ns, batch.shape[0], first_table.shape[1])
    return jnp.transpose(x, (1, 0, 2))


# speedup vs baseline: 1.0023x; 1.0023x over previous
import functools

import jax
import jax.numpy as jnp
from jax import lax
from jax.experimental import pallas as pl
from jax.experimental.pallas import tpu as pltpu
from jax.experimental.pallas import tpu_sc as plsc

L = 16
NC = 2
NS = 16
NW = NC * NS
G = 128
NB = 8


@functools.partial(jax.jit, static_argnums=(3, 4))
def _lookup(idx, first_table, second_table, n_per_w, slice_rows):
    dim = first_table.shape[1]
    n_total = idx.shape[0]
    ns = 50
    nb = n_total // ns
    buf_n = n_per_w + G

    mesh = plsc.VectorSubcoreMesh(core_axis_name="c", subcore_axis_name="s")

    @functools.partial(
        pl.kernel,
        out_type=jax.ShapeDtypeStruct((n_total, dim), jnp.float32),
        mesh=mesh,
        compiler_params=pltpu.CompilerParams(
            use_tc_tiling_on_sc=False, needs_layout_passes=False),
        scratch_types=dict(
            idx_v=pltpu.VMEM((n_per_w,), jnp.int32),
            idx1_v=pltpu.VMEM((buf_n,), jnp.int32),
            pos1_v=pltpu.VMEM((buf_n,), jnp.int32),
            idx2_v=pltpu.VMEM((buf_n,), jnp.int32),
            pos2_v=pltpu.VMEM((buf_n,), jnp.int32),
            rows_v=pltpu.VMEM((NB, G, dim), jnp.float32),
            sem_g=pltpu.SemaphoreType.DMA((NB,)),
            sem_s=pltpu.SemaphoreType.DMA((NB,)),
        ),
    )
    def k(idx_hbm, ft_hbm, st_hbm, out_hbm, *, idx_v, idx1_v, pos1_v,
          idx2_v, pos2_v, rows_v, sem_g, sem_s):
        wid = lax.axis_index("s") * NC + lax.axis_index("c")
        base = wid * n_per_w
        pltpu.sync_copy(idx_hbm.at[pl.ds(base, n_per_w)], idx_v)
        lanes = lax.iota(jnp.int32, L)

        trash = jnp.full((L,), buf_n - 1, jnp.int32)

        def half(i, o1, k):
            v = idx_v[pl.ds(i * 2 * L + k * L, L)]
            m1 = v < slice_rows
            c1 = plsc.cumsum(m1.astype(jnp.int32))
            f = base + i * 2 * L + k * L + lanes
            fb = f // ns
            p = (f - fb * ns) * nb + fb
            off1 = jnp.where(m1, o1 + c1 - 1, trash)
            off2 = jnp.where(m1, trash, (i * 2 * L + k * L - o1)
                             + (lanes - c1))
            plsc.store_scatter(idx1_v, [off1], v)
            plsc.store_scatter(pos1_v, [off1], p)
            plsc.store_scatter(idx2_v, [off2], v - slice_rows)
            plsc.store_scatter(pos2_v, [off2], p)
            return o1 + c1[L - 1]

        def step(i, o1):
            o1 = half(i, o1, 0)
            return half(i, o1, 1)

        n1 = lax.fori_loop(0, n_per_w // (2 * L), step, jnp.int32(0))
        n2 = n_per_w - n1

        def pad_dups(n, i_ref, p_ref):
            @pl.when(n > 0)
            def _():
                last = jnp.full((L,), n - 1, jnp.int32)
                li = plsc.load_gather(i_ref, [last])
                lp = plsc.load_gather(p_ref, [last])
                for t in range(G // L):
                    i_ref[pl.ds(n + t * L, L)] = li
                    p_ref[pl.ds(n + t * L, L)] = lp

        pad_dups(n1, idx1_v, pos1_v)
        pad_dups(n2, idx2_v, pos2_v)

        c1n = (n1 + G - 1) // G
        ct = c1n + (n2 + G - 1) // G

        def issue_gather(t, b):
            @pl.when(t < c1n)
            def _():
                pltpu.async_copy(
                    ft_hbm.at[idx1_v.at[pl.ds(t * G, G)]],
                    rows_v.at[b], sem_g.at[b])

            @pl.when(jnp.logical_and(t >= c1n, t < ct))
            def _():
                pltpu.async_copy(
                    st_hbm.at[idx2_v.at[pl.ds((t - c1n) * G, G)]],
                    rows_v.at[b], sem_g.at[b])

        def issue_scatter(t, b):
            @pl.when(t < c1n)
            def _():
                pltpu.async_copy(
                    rows_v.at[b], out_hbm.at[pos1_v.at[pl.ds(t * G, G)]],
                    sem_s.at[b])

            @pl.when(jnp.logical_and(t >= c1n, t < ct))
            def _():
                pltpu.async_copy(
                    rows_v.at[b],
                    out_hbm.at[pos2_v.at[pl.ds((t - c1n) * G, G)]],
                    sem_s.at[b])

        def drain(sem_b):
            pltpu.make_async_copy(
                ft_hbm.at[pl.ds(0, G)], rows_v.at[0], sem_b).wait()

        for b in range(NB):
            issue_gather(jnp.int32(b), b)

        def round_body(r, _):
            t0 = r * NB
            for b in range(NB):
                t = t0 + b

                @pl.when(t < ct)
                def _():
                    drain(sem_g.at[b])
                issue_scatter(t, b)
            for b in range(NB):
                tn = t0 + b + NB

                @pl.when(tn < ct)
                def _():
                    drain(sem_s.at[b])
                issue_gather(tn, b)
            return 0

        rounds = (ct + NB - 1) // NB
        lax.fori_loop(0, rounds, round_body, 0)

        for b in range(NB):
            @pl.when(b < ct)
            def _():
                drain(sem_s.at[b])

    return k(idx, first_table, second_table)


def kernel(batch, first_table, second_table):
    idx = batch.reshape(-1).astype(jnp.int32)
    n_total = idx.shape[0]
    assert n_total % (NW * L) == 0
    out = _lookup(idx, first_table, second_table, n_total // NW,
                  int(first_table.shape[0]))
    ns = batch.shape[1]
    x = out.reshape(ns, batch.shape[0], first_table.shape[1])
    return jnp.transpose(x, (1, 0, 2))


# speedup vs baseline: 1.0104x; 1.0081x over previous
import functools

import jax
import jax.numpy as jnp
from jax import lax
from jax.experimental import pallas as pl
from jax.experimental.pallas import tpu as pltpu
from jax.experimental.pallas import tpu_sc as plsc

L = 16
NC = 2
NS = 16
NW = NC * NS
G = 128
NB = 8


@functools.partial(jax.jit, static_argnums=(3, 4))
def _lookup(idx, first_table, second_table, n_per_w, slice_rows):
    dim = first_table.shape[1]
    n_total = idx.shape[0]
    ns = 50
    nb = n_total // ns
    buf_n = n_per_w + G

    mesh = plsc.VectorSubcoreMesh(core_axis_name="c", subcore_axis_name="s")

    @functools.partial(
        pl.kernel,
        out_type=jax.ShapeDtypeStruct((n_total, dim), jnp.float32),
        mesh=mesh,
        compiler_params=pltpu.CompilerParams(
            use_tc_tiling_on_sc=False, needs_layout_passes=False),
        scratch_types=dict(
            idx_v=pltpu.VMEM((n_per_w,), jnp.int32),
            idx1_v=pltpu.VMEM((buf_n,), jnp.int32),
            pos1_v=pltpu.VMEM((buf_n,), jnp.int32),
            idx2_v=pltpu.VMEM((buf_n,), jnp.int32),
            pos2_v=pltpu.VMEM((buf_n,), jnp.int32),
            rows_v=pltpu.VMEM((NB, G, dim), jnp.float32),
            sem_g=pltpu.SemaphoreType.DMA((NB,)),
            sem_s=pltpu.SemaphoreType.DMA((NB,)),
        ),
    )
    def k(idx_hbm, ft_hbm, st_hbm, out_hbm, *, idx_v, idx1_v, pos1_v,
          idx2_v, pos2_v, rows_v, sem_g, sem_s):
        wid = lax.axis_index("s") * NC + lax.axis_index("c")
        base = wid * n_per_w
        pltpu.sync_copy(idx_hbm.at[pl.ds(base, n_per_w)], idx_v)
        lanes = lax.iota(jnp.int32, L)

        trash = jnp.full((L,), buf_n - 1, jnp.int32)

        def half(i, o1, k):
            v = idx_v[pl.ds(i * 2 * L + k * L, L)]
            m1 = v < slice_rows
            c1 = plsc.cumsum(m1.astype(jnp.int32))
            f = base + i * 2 * L + k * L + lanes
            fb = f // ns
            p = (f - fb * ns) * nb + fb
            off1 = jnp.where(m1, o1 + c1 - 1, trash)
            off2 = jnp.where(m1, trash, (i * 2 * L + k * L - o1)
                             + (lanes - c1))
            plsc.store_scatter(idx1_v, [off1], v)
            plsc.store_scatter(pos1_v, [off1], p)
            plsc.store_scatter(idx2_v, [off2], v - slice_rows)
            plsc.store_scatter(pos2_v, [off2], p)
            return o1 + c1[L - 1]

        def step(i, o1):
            o1 = half(i, o1, 0)
            return half(i, o1, 1)

        nsteps = n_per_w // (2 * L)
        k1 = nsteps // 3
        o1p = lax.fori_loop(0, k1, step, jnp.int32(0))
        for b in range(NB):
            @pl.when((b + 1) * G <= o1p)
            def _():
                pltpu.async_copy(
                    ft_hbm.at[idx1_v.at[pl.ds(b * G, G)]],
                    rows_v.at[b], sem_g.at[b])

        n1 = lax.fori_loop(k1, nsteps, step, o1p)
        n2 = n_per_w - n1

        def pad_dups(n, i_ref, p_ref):
            @pl.when(n > 0)
            def _():
                last = jnp.full((L,), n - 1, jnp.int32)
                li = plsc.load_gather(i_ref, [last])
                lp = plsc.load_gather(p_ref, [last])
                for t in range(G // L):
                    i_ref[pl.ds(n + t * L, L)] = li
                    p_ref[pl.ds(n + t * L, L)] = lp

        pad_dups(n1, idx1_v, pos1_v)
        pad_dups(n2, idx2_v, pos2_v)

        c1n = (n1 + G - 1) // G
        ct = c1n + (n2 + G - 1) // G

        def issue_gather(t, b):
            @pl.when(t < c1n)
            def _():
                pltpu.async_copy(
                    ft_hbm.at[idx1_v.at[pl.ds(t * G, G)]],
                    rows_v.at[b], sem_g.at[b])

            @pl.when(jnp.logical_and(t >= c1n, t < ct))
            def _():
                pltpu.async_copy(
                    st_hbm.at[idx2_v.at[pl.ds((t - c1n) * G, G)]],
                    rows_v.at[b], sem_g.at[b])

        def issue_scatter(t, b):
            @pl.when(t < c1n)
            def _():
                pltpu.async_copy(
                    rows_v.at[b], out_hbm.at[pos1_v.at[pl.ds(t * G, G)]],
                    sem_s.at[b])

            @pl.when(jnp.logical_and(t >= c1n, t < ct))
            def _():
                pltpu.async_copy(
                    rows_v.at[b],
                    out_hbm.at[pos2_v.at[pl.ds((t - c1n) * G, G)]],
                    sem_s.at[b])

        def drain(sem_b):
            pltpu.make_async_copy(
                ft_hbm.at[pl.ds(0, G)], rows_v.at[0], sem_b).wait()

        for b in range(NB):
            @pl.when((b + 1) * G > o1p)
            def _():
                issue_gather(jnp.int32(b), b)

        def round_body(r, _):
            t0 = r * NB
            for b in range(NB):
                t = t0 + b

                @pl.when(t < ct)
                def _():
                    drain(sem_g.at[b])
                issue_scatter(t, b)
            for b in range(NB):
                tn = t0 + b + NB

                @pl.when(tn < ct)
                def _():
                    drain(sem_s.at[b])
                issue_gather(tn, b)
            return 0

        rounds = (ct + NB - 1) // NB
        lax.fori_loop(0, rounds, round_body, 0)

        for b in range(NB):
            @pl.when(b < ct)
            def _():
                drain(sem_s.at[b])

    return k(idx, first_table, second_table)


def kernel(batch, first_table, second_table):
    idx = batch.reshape(-1).astype(jnp.int32)
    n_total = idx.shape[0]
    assert n_total % (NW * L) == 0
    out = _lookup(idx, first_table, second_table, n_total // NW,
                  int(first_table.shape[0]))
    ns = batch.shape[1]
    x = out.reshape(ns, batch.shape[0], first_table.shape[1])
    return jnp.transpose(x, (1, 0, 2))
